# baseline (speedup 1.0000x reference)
import functools

import jax
import jax.numpy as jnp
from jax import lax
from jax.experimental import pallas as pl
from jax.experimental.pallas import tpu as pltpu
from jax.experimental.pallas import tpu_sc as plsc

EMB_DIM = 64
NUM_CORES = 2
NUM_SUBCORES = 16
NUM_WORKERS = NUM_CORES * NUM_SUBCORES

CHUNK = 128
NBUF = 8


def _make_emb_kernel(B: int, D: int):
  b_per_w = B // NUM_WORKERS
  n_chunks = b_per_w // CHUNK
  assert B % NUM_WORKERS == 0 and b_per_w % CHUNK == 0
  assert n_chunks % NBUF == 0 and n_chunks // NBUF >= 2
  n_rings = n_chunks // NBUF
  mesh = plsc.VectorSubcoreMesh(core_axis_name="c", subcore_axis_name="s")

  @functools.partial(
      pl.kernel,
      mesh=mesh,
      out_type=jax.ShapeDtypeStruct((B, D), jnp.float32),
      compiler_params=pltpu.CompilerParams(use_tc_tiling_on_sc=False),
      scratch_types=[
          pltpu.VMEM((n_chunks, CHUNK), jnp.int32),
          [pltpu.VMEM((CHUNK, D), jnp.float32) for _ in range(NBUF)],
          [pltpu.SemaphoreType.DMA for _ in range(NBUF)],
          [pltpu.SemaphoreType.DMA for _ in range(NBUF)],
      ],
  )
  def emb(idx_hbm, table_hbm, out_hbm, idx_v, rows, sem_g, sem_o):
    wid = lax.axis_index("s") * NUM_CORES + lax.axis_index("c")
    base = wid * b_per_w

    pltpu.sync_copy(idx_hbm.at[wid], idx_v)

    def gather_desc(j, b):
      return pltpu.make_async_copy(
          table_hbm.at[idx_v.at[j]], rows[b], sem_g[b])

    def put_desc(j, b):
      return pltpu.make_async_copy(
          rows[b], out_hbm.at[pl.ds(base + j * CHUNK, CHUNK)], sem_o[b])

    for b in range(NBUF - 1):
      gather_desc(b, b).start()

    def step(j, k, first, last):
      fb = (k + NBUF - 1) % NBUF
      if not first:
        put_desc(j - 1, fb).wait()
      if not last:
        gather_desc(j + NBUF - 1, fb).start()
      gather_desc(j, k).wait()
      put_desc(j, k).start()

    for k in range(NBUF):
      step(k, k, first=(k == 0), last=False)

    def ring(r, carry):
      j0 = r * NBUF
      for k in range(NBUF):
        step(j0 + k, k, first=False, last=False)
      return carry

    lax.fori_loop(1, n_rings - 1, ring, 0)

    j0 = (n_rings - 1) * NBUF
    for k in range(NBUF):
      step(j0 + k, k, first=False, last=(k != 0))

    put_desc(j0 + NBUF - 1, NBUF - 1).wait()

  return emb


@jax.jit
def kernel(input, W):
  D = W.shape[1]
  idx = input.reshape(-1).astype(jnp.int32)
  B = idx.shape[0]
  b_per_w = B // NUM_WORKERS
  idx3 = idx.reshape(NUM_WORKERS, b_per_w // CHUNK, CHUNK)
  emb = _make_emb_kernel(B, D)
  out = emb(idx3, W)
  return out.reshape(input.shape + (D,))


# speedup vs baseline: 1.0022x; 1.0022x over previous
import functools

import jax
import jax.numpy as jnp
from jax import lax
from jax.experimental import pallas as pl
from jax.experimental.pallas import tpu as pltpu
from jax.experimental.pallas import tpu_sc as plsc

NUM_CORES = 2
NUM_SUBCORES = 16
NUM_WORKERS = NUM_CORES * NUM_SUBCORES

NBUF = 4


def _make_emb_kernel(BATCH: int, SEQ: int, D: int):
  rows_per_w = BATCH // NUM_WORKERS
  assert BATCH % NUM_WORKERS == 0 and rows_per_w % NBUF == 0
  assert rows_per_w // NBUF >= 2
  n_rings = rows_per_w // NBUF
  s0 = min(128, SEQ)
  splits = [(0, s0)]
  if SEQ > 128:
    assert SEQ <= 256 and s0 % 8 == 0
    splits.append((s0, SEQ - s0))
  mesh = plsc.VectorSubcoreMesh(core_axis_name="c", subcore_axis_name="s")

  @functools.partial(
      pl.kernel,
      mesh=mesh,
      out_type=jax.ShapeDtypeStruct((BATCH, SEQ, D), jnp.float32),
      compiler_params=pltpu.CompilerParams(use_tc_tiling_on_sc=False),
      scratch_types=[
          pltpu.VMEM((rows_per_w, SEQ), jnp.int32),
          [pltpu.VMEM((SEQ, D), jnp.float32) for _ in range(NBUF)],
          [pltpu.SemaphoreType.DMA for _ in range(NBUF)],
          [pltpu.SemaphoreType.DMA for _ in range(NBUF)],
      ],
  )
  def emb(idx_hbm, table_hbm, out_hbm, idx_v, rows, sem_g, sem_o):
    wid = lax.axis_index("s") * NUM_CORES + lax.axis_index("c")
    base = wid * rows_per_w

    pltpu.sync_copy(idx_hbm.at[pl.ds(base, rows_per_w)], idx_v)

    def gather_descs(b, k):
      return [
          pltpu.make_async_copy(
              table_hbm.at[idx_v.at[b, pl.ds(off, ln)]],
              rows[k].at[pl.ds(off, ln)],
              sem_g[k])
          for off, ln in splits
      ]

    def put_desc(b, k):
      return pltpu.make_async_copy(rows[k], out_hbm.at[base + b], sem_o[k])

    def start_gathers(b, k):
      for d in gather_descs(b, k):
        d.start()

    def wait_gathers(b, k):
      for d in gather_descs(b, k):
        d.wait()

    for k in range(NBUF - 1):
      start_gathers(k, k)

    def step(b, k, first, last):
      fb = (k + NBUF - 1) % NBUF
      if not first:
        put_desc(b - 1, fb).wait()
      if not last:
        start_gathers(b + NBUF - 1, fb)
      wait_gathers(b, k)
      put_desc(b, k).start()

    for k in range(NBUF):
      step(k, k, first=(k == 0), last=False)

    def ring(r, carry):
      b0 = r * NBUF
      for k in range(NBUF):
        step(b0 + k, k, first=False, last=False)
      return carry

    lax.fori_loop(1, n_rings - 1, ring, 0)

    b0 = (n_rings - 1) * NBUF
    for k in range(NBUF):
      step(b0 + k, k, first=False, last=(k != 0))

    put_desc(b0 + NBUF - 1, NBUF - 1).wait()

  return emb


@jax.jit
def kernel(input, W):
  idx = input.astype(jnp.int32)
  emb = _make_emb_kernel(idx.shape[0], idx.shape[1], W.shape[1])
  return emb(idx, W)
